# baseline (speedup 1.0000x reference)
import functools

import jax
import jax.numpy as jnp
from jax import lax
from jax.experimental import pallas as pl
from jax.experimental.pallas import tpu as pltpu
from jax.experimental.pallas import tpu_sc as plsc

B, V, J, K = 512, 10475, 55, 21
JK = J + K
NC, NS = 2, 16
NW = NC * NS


def _sc_gather(vT, sT, jT, idxc):
    mesh = plsc.VectorSubcoreMesh(core_axis_name="c", subcore_axis_name="s")

    @functools.partial(
        pl.kernel,
        out_type=jax.ShapeDtypeStruct((JK, 4, 4, B), jnp.float32),
        mesh=mesh,
        compiler_params=pltpu.CompilerParams(
            needs_layout_passes=False, use_tc_tiling_on_sc=True),
        scratch_types=[
            pltpu.VMEM((K, 24), jnp.int32),
            pltpu.VMEM((3, 8, B), jnp.float32),
            pltpu.VMEM((4, 4, B), jnp.float32),
            pltpu.SemaphoreType.DMA,
        ],
    )
    def k(vT_hbm, sT_hbm, jT_hbm, idxc_hbm, oT_hbm,
          idxc_v, vrows_v, slab_v, sem):
        w = lax.axis_index("s") * NC + lax.axis_index("c")

        @pl.when(w >= K)
        def _():
            base = (w - K) * 5
            pltpu.async_copy(jT_hbm.at[pl.ds(base, 5)],
                             oT_hbm.at[pl.ds(base, 5)], sem).wait()

        @pl.when(w < K)
        def _():
            j = w
            pltpu.sync_copy(idxc_hbm.at[j], idxc_v.at[j])
            i = idxc_v[j, pl.ds(0, 16)][0]
            gcps = [pltpu.async_copy(sT_hbm.at[i], slab_v, sem)]
            for c in range(3):
                gcps.append(pltpu.async_copy(
                    vT_hbm.at[c].at[idxc_v.at[j, pl.ds(16, 8)]],
                    vrows_v.at[c], sem))
            for cp in gcps:
                cp.wait()
            for r in range(3):
                for g in range(B // 16):
                    slab_v[r, 3, pl.ds(g * 16, 16)] = (
                        vrows_v[r, 0, pl.ds(g * 16, 16)])
            pltpu.sync_copy(slab_v, oT_hbm.at[J + j])

    return k(vT, sT, jT, idxc)


def kernel(vertices, joints_transforms, skinning_transforms, extra_joints_idxs):
    idx32 = extra_joints_idxs.astype(jnp.int32)
    idxc = jnp.broadcast_to(idx32[:, None], (K, 24))
    oT = _sc_gather(
        vertices.transpose(2, 1, 0),
        skinning_transforms.transpose(1, 2, 3, 0),
        joints_transforms.transpose(1, 2, 3, 0),
        idxc,
    )
    return oT.transpose(3, 0, 1, 2)


# speedup vs baseline: 3.1235x; 3.1235x over previous
import functools

import jax
import jax.numpy as jnp
from jax import lax
from jax.experimental import pallas as pl
from jax.experimental.pallas import tpu as pltpu
from jax.experimental.pallas import tpu_sc as plsc

B, V, J, K = 512, 10475, 55, 21
JK = J + K
NC, NS = 2, 16
NW = NC * NS


def _sc_gather(vT, sT, jT, idxc):
    mesh = plsc.VectorSubcoreMesh(core_axis_name="c", subcore_axis_name="s")

    @functools.partial(
        pl.kernel,
        out_type=jax.ShapeDtypeStruct((JK, 4, 4, B), jnp.float32),
        mesh=mesh,
        compiler_params=pltpu.CompilerParams(
            needs_layout_passes=False, use_tc_tiling_on_sc=True),
        scratch_types=[
            pltpu.VMEM((K, 24), jnp.int32),
            pltpu.VMEM((3, 8, B), jnp.float32),
            pltpu.VMEM((4, 4, B), jnp.float32),
            pltpu.VMEM((5, 4, 4, B), jnp.float32),
            pltpu.SemaphoreType.DMA,
        ],
    )
    def k(vT_hbm, sT_hbm, jT_hbm, idxc_hbm, oT_hbm,
          idxc_v, vrows_v, slab_v, jstage_v, sem):
        w = lax.axis_index("s") * NC + lax.axis_index("c")

        @pl.when(w >= K)
        def _():
            base = (w - K) * 5
            pltpu.async_copy(jT_hbm.at[pl.ds(base, 5)], jstage_v, sem).wait()
            pltpu.async_copy(jstage_v, oT_hbm.at[pl.ds(base, 5)], sem).wait()

        @pl.when(w < K)
        def _():
            j = w
            pltpu.sync_copy(idxc_hbm.at[j], idxc_v.at[j])
            i = idxc_v[j, pl.ds(0, 16)][0]
            gcps = [pltpu.async_copy(sT_hbm.at[i], slab_v, sem)]
            for c in range(3):
                gcps.append(pltpu.async_copy(
                    vT_hbm.at[c].at[idxc_v.at[j, pl.ds(16, 8)]],
                    vrows_v.at[c], sem))
            for cp in gcps:
                cp.wait()
            for r in range(3):
                for g in range(B // 16):
                    slab_v[r, 3, pl.ds(g * 16, 16)] = (
                        vrows_v[r, 0, pl.ds(g * 16, 16)])
            pltpu.sync_copy(slab_v, oT_hbm.at[J + j])

    return k(vT, sT, jT, idxc)


def kernel(vertices, joints_transforms, skinning_transforms, extra_joints_idxs):
    idx32 = extra_joints_idxs.astype(jnp.int32)
    idxc = jnp.broadcast_to(idx32[:, None], (K, 24))
    oT = _sc_gather(
        vertices.transpose(2, 1, 0),
        skinning_transforms.transpose(1, 2, 3, 0),
        joints_transforms.transpose(1, 2, 3, 0),
        idxc,
    )
    return oT.transpose(3, 0, 1, 2)
